# baseline (speedup 1.0000x reference)
import functools

import jax
import jax.numpy as jnp
from jax import lax
from jax.experimental import pallas as pl
from jax.experimental.pallas import tpu as pltpu
from jax.experimental.pallas import tpu_sc as plsc

_CHUNK = 32
_NBUF = 3
_LOOK = _NBUF - 2
_ROWS_C0 = 288
_ROWS_C1 = 224


def _ring(batch, n_chunks, base, table_hbm, out_hbm, bufs, rsems, wsems):
    reads = [None] * _NBUF
    writes = [[] for _ in range(_NBUF)]
    for c in range(min(_LOOK, n_chunks)):
        reads[c % _NBUF] = pltpu.async_copy(
            table_hbm.at[pl.ds(base + c * _CHUNK, _CHUNK)], bufs[c % _NBUF], rsems[c % _NBUF])
    for c in range(n_chunks):
        slot = c % _NBUF
        nc = c + _LOOK
        if nc < n_chunks:
            slot_n = nc % _NBUF
            for h in writes[slot_n]:
                h.wait()
            writes[slot_n] = []
            reads[slot_n] = pltpu.async_copy(
                table_hbm.at[pl.ds(base + nc * _CHUNK, _CHUNK)], bufs[slot_n], rsems[slot_n])
        reads[slot].wait()
        lo = base + c * _CHUNK
        for b in range(batch):
            writes[slot].append(
                pltpu.async_copy(bufs[slot], out_hbm.at[b, pl.ds(lo, _CHUNK)], wsems[slot]))
    for lst in writes:
        for h in lst:
            h.wait()


def _sc_body(batch, n_subcores, table_hbm, out_hbm, *scratch):
    bufs = scratch[:_NBUF]
    rsems = scratch[_NBUF:2 * _NBUF]
    wsems = scratch[2 * _NBUF:]
    cidx = lax.axis_index("c")
    sidx = lax.axis_index("s")

    @pl.when(cidx == 0)
    def _():
        _ring(batch, _ROWS_C0 // _CHUNK, sidx * _ROWS_C0,
              table_hbm, out_hbm, bufs, rsems, wsems)

    @pl.when(cidx == 1)
    def _():
        _ring(batch, _ROWS_C1 // _CHUNK, n_subcores * _ROWS_C0 + sidx * _ROWS_C1,
              table_hbm, out_hbm, bufs, rsems, wsems)


def kernel(x, position_embeddings):
    batch = x.shape[0]
    seq_len = x.shape[1]
    n_rows, dim = position_embeddings.shape
    info = plsc.get_sparse_core_info()
    assert info.num_subcores * (_ROWS_C0 + _ROWS_C1) == seq_len
    mesh = plsc.VectorSubcoreMesh(core_axis_name="c", subcore_axis_name="s")
    body = functools.partial(_sc_body, batch, info.num_subcores)
    run = pl.kernel(
        body,
        out_type=jax.ShapeDtypeStruct((batch, seq_len, dim), position_embeddings.dtype),
        mesh=mesh,
        scratch_types=(
            [pltpu.VMEM((_CHUNK, dim), jnp.float32) for _ in range(_NBUF)]
            + [pltpu.SemaphoreType.DMA for _ in range(2 * _NBUF)]
        ),
    )
    return run(position_embeddings)


# speedup vs baseline: 1.0085x; 1.0085x over previous
import functools

import jax
import jax.numpy as jnp
from jax import lax
from jax.experimental import pallas as pl
from jax.experimental.pallas import tpu as pltpu
from jax.experimental.pallas import tpu_sc as plsc

_CHUNK = 32
_NBUF = 3
_LOOK = _NBUF - 2
_ROWS_C0 = 224
_ROWS_C1 = 288


def _ring(batch, n_chunks, base, table_hbm, out_hbm, bufs, rsems, wsems):
    reads = [None] * _NBUF
    writes = [[] for _ in range(_NBUF)]
    for c in range(min(_LOOK, n_chunks)):
        reads[c % _NBUF] = pltpu.async_copy(
            table_hbm.at[pl.ds(base + c * _CHUNK, _CHUNK)], bufs[c % _NBUF], rsems[c % _NBUF])
    for c in range(n_chunks):
        slot = c % _NBUF
        nc = c + _LOOK
        if nc < n_chunks:
            slot_n = nc % _NBUF
            for h in writes[slot_n]:
                h.wait()
            writes[slot_n] = []
            reads[slot_n] = pltpu.async_copy(
                table_hbm.at[pl.ds(base + nc * _CHUNK, _CHUNK)], bufs[slot_n], rsems[slot_n])
        reads[slot].wait()
        lo = base + c * _CHUNK
        for b in range(batch):
            writes[slot].append(
                pltpu.async_copy(bufs[slot], out_hbm.at[b, pl.ds(lo, _CHUNK)], wsems[slot]))
    for lst in writes:
        for h in lst:
            h.wait()


def _sc_body(batch, n_subcores, table_hbm, out_hbm, *scratch):
    bufs = scratch[:_NBUF]
    rsems = scratch[_NBUF:2 * _NBUF]
    wsems = scratch[2 * _NBUF:]
    cidx = lax.axis_index("c")
    sidx = lax.axis_index("s")

    @pl.when(cidx == 0)
    def _():
        _ring(batch, _ROWS_C0 // _CHUNK, sidx * _ROWS_C0,
              table_hbm, out_hbm, bufs, rsems, wsems)

    @pl.when(cidx == 1)
    def _():
        _ring(batch, _ROWS_C1 // _CHUNK, n_subcores * _ROWS_C0 + sidx * _ROWS_C1,
              table_hbm, out_hbm, bufs, rsems, wsems)


def kernel(x, position_embeddings):
    batch = x.shape[0]
    seq_len = x.shape[1]
    n_rows, dim = position_embeddings.shape
    info = plsc.get_sparse_core_info()
    assert info.num_subcores * (_ROWS_C0 + _ROWS_C1) == seq_len
    mesh = plsc.VectorSubcoreMesh(core_axis_name="c", subcore_axis_name="s")
    body = functools.partial(_sc_body, batch, info.num_subcores)
    run = pl.kernel(
        body,
        out_type=jax.ShapeDtypeStruct((batch, seq_len, dim), position_embeddings.dtype),
        mesh=mesh,
        scratch_types=(
            [pltpu.VMEM((_CHUNK, dim), jnp.float32) for _ in range(_NBUF)]
            + [pltpu.SemaphoreType.DMA for _ in range(2 * _NBUF)]
        ),
    )
    return run(position_embeddings)


# speedup vs baseline: 1.0608x; 1.0519x over previous
import functools

import jax
import jax.numpy as jnp
from jax import lax
from jax.experimental import pallas as pl
from jax.experimental.pallas import tpu as pltpu
from jax.experimental.pallas import tpu_sc as plsc

_CHUNK = 32
_NBUF = 3
_LOOK = _NBUF - 2


def _sc_body(batch, rows_per_w, n_chunks, n_cores,
             table_hbm, out_hbm, *scratch):
    bufs = scratch[:_NBUF]
    rsems = scratch[_NBUF:2 * _NBUF]
    wsems = scratch[2 * _NBUF:]
    wid = lax.axis_index("s") * n_cores + lax.axis_index("c")
    base = wid * rows_per_w

    reads = [None] * _NBUF
    writes = [[] for _ in range(_NBUF)]
    for c in range(min(_LOOK, n_chunks)):
        reads[c % _NBUF] = pltpu.async_copy(
            table_hbm.at[pl.ds(base + c * _CHUNK, _CHUNK)], bufs[c % _NBUF], rsems[c % _NBUF])
    for c in range(n_chunks):
        slot = c % _NBUF
        nc = c + _LOOK
        if nc < n_chunks:
            slot_n = nc % _NBUF
            for h in writes[slot_n]:
                h.wait()
            writes[slot_n] = []
            reads[slot_n] = pltpu.async_copy(
                table_hbm.at[pl.ds(base + nc * _CHUNK, _CHUNK)], bufs[slot_n], rsems[slot_n])
        reads[slot].wait()
        lo = base + c * _CHUNK
        for b in range(batch):
            writes[slot].append(
                pltpu.async_copy(bufs[slot], out_hbm.at[b, pl.ds(lo, _CHUNK)], wsems[slot]))
    for lst in writes:
        for h in lst:
            h.wait()


def kernel(x, position_embeddings):
    batch = x.shape[0]
    seq_len = x.shape[1]
    n_rows, dim = position_embeddings.shape
    info = plsc.get_sparse_core_info()
    n_workers = info.num_cores * info.num_subcores
    rows_per_w = seq_len // n_workers
    n_chunks = rows_per_w // _CHUNK
    mesh = plsc.VectorSubcoreMesh(core_axis_name="c", subcore_axis_name="s")
    body = functools.partial(_sc_body, batch, rows_per_w, n_chunks, info.num_cores)
    run = pl.kernel(
        body,
        out_type=jax.ShapeDtypeStruct((batch, seq_len, dim), position_embeddings.dtype),
        mesh=mesh,
        scratch_types=(
            [pltpu.VMEM((_CHUNK, dim), jnp.float32) for _ in range(_NBUF)]
            + [pltpu.SemaphoreType.DMA for _ in range(2 * _NBUF)]
        ),
    )
    return run(position_embeddings)
